# Initial kernel scaffold; baseline (speedup 1.0000x reference)
import functools

import jax
import jax.numpy as jnp
from jax import lax
from jax.experimental import pallas as pl
from jax.experimental.pallas import tpu as pltpu
from jax.experimental.pallas import tpu_sc as plsc

B = 4096
NCAT = 100
NCON = 26
DIM = 64
NW = 32
BPW = B // NW
NPAD = 112
L = 16


def _body(xc, xn, table, w, bias, out, idx_v, rows_v, num_v, xn_v, w_v, b_v,
          sem):
    wid = lax.axis_index("s") * 2 + lax.axis_index("c")
    b0 = wid * BPW

    pltpu.sync_copy(xc.at[pl.ds(b0, BPW), :], idx_v.at[:, pl.ds(0, NCAT)])
    pltpu.sync_copy(xn.at[pl.ds(b0, BPW), :], xn_v)
    pltpu.sync_copy(w, w_v)
    pltpu.sync_copy(bias, b_v)

    lanes = lax.iota(jnp.int32, L)

    def idx_body(i, carry):
        for c in range(NPAD // L):
            v = idx_v[i, pl.ds(c * L, L)]
            off = lanes * 1000 + (2 + 1000 * L * c)
            if (c + 1) * L <= NCAT:
                res = v + off
            else:
                res = jnp.where(lanes < NCAT - c * L, v + off, 0)
            idx_v[i, pl.ds(c * L, L)] = res
        return carry

    lax.fori_loop(0, BPW, idx_body, 0)

    def main_body(i, carry):
        pltpu.async_copy(table.at[idx_v.at[i]], rows_v, sem).wait()

        ii = jnp.full((L,), i, dtype=jnp.int32)
        for j in range(NCON):
            jj = jnp.full((L,), j, dtype=jnp.int32)
            xb = plsc.load_gather(xn_v, [ii, jj])
            for d in range(DIM // L):
                num_v[j, pl.ds(d * L, L)] = (
                    xb * w_v[j, pl.ds(d * L, L)] + b_v[j, pl.ds(d * L, L)])

        bg = b0 + i
        pltpu.sync_copy(rows_v.at[pl.ds(0, NCAT)], out.at[bg, pl.ds(0, NCAT), :])
        pltpu.sync_copy(num_v, out.at[bg, pl.ds(NCAT, NCON), :])
        return carry

    lax.fori_loop(0, BPW, main_body, 0)


_mesh = plsc.VectorSubcoreMesh(core_axis_name="c", subcore_axis_name="s")

_sc_call = functools.partial(
    pl.kernel,
    out_type=jax.ShapeDtypeStruct((B, NCAT + NCON, DIM), jnp.float32),
    mesh=_mesh,
    scratch_types=[
        pltpu.VMEM((BPW, NPAD), jnp.int32),
        pltpu.VMEM((NPAD, DIM), jnp.float32),
        pltpu.VMEM((NCON, DIM), jnp.float32),
        pltpu.VMEM((BPW, NCON), jnp.float32),
        pltpu.VMEM((NCON, DIM), jnp.float32),
        pltpu.VMEM((NCON, DIM), jnp.float32),
        pltpu.SemaphoreType.DMA,
    ],
)(_body)


@jax.jit
def kernel(x_categ, x_numer, embed_table, num_weight, num_bias):
    return _sc_call(
        x_categ.astype(jnp.int32),
        x_numer.astype(jnp.float32),
        embed_table.astype(jnp.float32),
        num_weight.astype(jnp.float32),
        num_bias.astype(jnp.float32),
    )


# speedup vs baseline: 1.3555x; 1.3555x over previous
import functools

import jax
import jax.numpy as jnp
from jax import lax
from jax.experimental import pallas as pl
from jax.experimental.pallas import tpu as pltpu
from jax.experimental.pallas import tpu_sc as plsc

B = 4096
NCAT = 100
NCON = 26
DIM = 64
NROW = NCAT + NCON
NW = 32
BPW = B // NW
NPAD = 112
L = 16


def _body(xc, xn, table, w, bias, out, xc_v, idx_v, rows_v, xn_v, w_v, b_v,
          sem):
    wid = lax.axis_index("s") * 2 + lax.axis_index("c")
    b0 = wid * BPW

    pltpu.sync_copy(xc.at[pl.ds(b0 * NCAT, BPW * NCAT)],
                    xc_v.at[pl.ds(0, BPW * NCAT)])
    pltpu.sync_copy(xn.at[pl.ds(b0 * NCON, BPW * NCON)],
                    xn_v.at[pl.ds(0, BPW * NCON)])
    pltpu.sync_copy(w, w_v)
    pltpu.sync_copy(bias, b_v)

    lanes = lax.iota(jnp.int32, L)

    def idx_body(i, carry):
        src = i * NCAT
        dst = i * NPAD
        for c in range(NPAD // L):
            if (c + 1) * L <= NCAT:
                v = xc_v[pl.ds(src + c * L, L)]
                idx_v[pl.ds(dst + c * L, L)] = (
                    v + (lanes * 1000 + (2 + 1000 * L * c)))
            else:
                v = xc_v[pl.ds(src + c * L, L)]
                off = lanes * 1000 + (2 + 1000 * L * c)
                idx_v[pl.ds(dst + c * L, L)] = jnp.where(
                    lanes < NCAT - c * L, v + off, 0)
        return carry

    lax.fori_loop(0, BPW, idx_body, 0)

    def main_body(i, carry):
        pltpu.async_copy(table.at[idx_v.at[pl.ds(i * NPAD, NPAD)]],
                         rows_v.at[pl.ds(0, NPAD)], sem).wait()

        xrow = [xn_v[pl.ds(i * NCON, L)], xn_v[pl.ds(i * NCON + L, L)]]
        for j in range(NCON):
            xb = jnp.take_along_axis(
                xrow[j // L], jnp.full((L,), j % L, dtype=jnp.int32), axis=0)
            for d in range(DIM // L):
                rows_v[NCAT + j, pl.ds(d * L, L)] = (
                    xb * w_v[j, pl.ds(d * L, L)] + b_v[j, pl.ds(d * L, L)])

        pltpu.sync_copy(rows_v.at[pl.ds(0, NROW)], out.at[b0 + i])
        return carry

    lax.fori_loop(0, BPW, main_body, 0)


_mesh = plsc.VectorSubcoreMesh(core_axis_name="c", subcore_axis_name="s")

_sc_call = functools.partial(
    pl.kernel,
    out_type=jax.ShapeDtypeStruct((B, NROW, DIM), jnp.float32),
    mesh=_mesh,
    compiler_params=pltpu.CompilerParams(use_tc_tiling_on_sc=False),
    scratch_types=[
        pltpu.VMEM((BPW * NCAT + L,), jnp.int32),
        pltpu.VMEM((BPW * NPAD,), jnp.int32),
        pltpu.VMEM((128, DIM), jnp.float32),
        pltpu.VMEM((BPW * NCON + L,), jnp.float32),
        pltpu.VMEM((NCON, DIM), jnp.float32),
        pltpu.VMEM((NCON, DIM), jnp.float32),
        pltpu.SemaphoreType.DMA,
    ],
)(_body)


@jax.jit
def kernel(x_categ, x_numer, embed_table, num_weight, num_bias):
    return _sc_call(
        x_categ.astype(jnp.int32).reshape(-1),
        x_numer.astype(jnp.float32).reshape(-1),
        embed_table.astype(jnp.float32),
        num_weight.astype(jnp.float32),
        num_bias.astype(jnp.float32),
    )


# speedup vs baseline: 4.5345x; 3.3452x over previous
import functools

import jax
import jax.numpy as jnp
from jax import lax
from jax.experimental import pallas as pl
from jax.experimental.pallas import tpu as pltpu
from jax.experimental.pallas import tpu_sc as plsc

B = 4096
NCAT = 100
NCON = 26
DIM = 64
NROW = NCAT + NCON
NW = 32
BPW = B // NW
NPAD = 112
L = 16
NBUF = 8
LAG = 4


def _body(xc, xn, table, w, bias, out, xc_v, idx_v, rows_v, xn_v, w_v, b_v,
          g_sem, o_sem):
    wid = lax.axis_index("s") * 2 + lax.axis_index("c")
    b0 = wid * BPW

    pltpu.sync_copy(xc.at[pl.ds(b0 * NCAT, BPW * NCAT)],
                    xc_v.at[pl.ds(0, BPW * NCAT)])
    pltpu.sync_copy(xn.at[pl.ds(b0 * NCON, BPW * NCON)],
                    xn_v.at[pl.ds(0, BPW * NCON)])
    pltpu.sync_copy(w, w_v)
    pltpu.sync_copy(bias, b_v)

    lanes = lax.iota(jnp.int32, L)

    def compute_idx(v):
        src = v * NCAT
        dst = v * NPAD
        for c in range(NPAD // L):
            vec = xc_v[pl.ds(src + c * L, L)]
            off = lanes * 1000 + (2 + 1000 * L * c)
            if (c + 1) * L <= NCAT:
                idx_v[pl.ds(dst + c * L, L)] = vec + off
            else:
                idx_v[pl.ds(dst + c * L, L)] = jnp.where(
                    lanes < NCAT - c * L, vec + off, 0)

    def issue_gather(v, k):
        pltpu.async_copy(table.at[idx_v.at[pl.ds(v * NPAD, NCAT)]],
                         rows_v.at[k, pl.ds(0, NCAT), :], g_sem.at[k])

    def wait_gather(k):
        pltpu.make_async_copy(
            table.at[pl.ds(0, NCAT), :],
            rows_v.at[k, pl.ds(0, NCAT), :], g_sem.at[k]).wait()

    def numeric(v, k):
        xrow = [xn_v[pl.ds(v * NCON, L)], xn_v[pl.ds(v * NCON + L, L)]]
        for j in range(NCON):
            xb = jnp.take_along_axis(
                xrow[j // L], jnp.full((L,), j % L, dtype=jnp.int32), axis=0)
            for d in range(DIM // L):
                rows_v[k, NCAT + j, pl.ds(d * L, L)] = (
                    xb * w_v[j, pl.ds(d * L, L)] + b_v[j, pl.ds(d * L, L)])

    def issue_out(v, k):
        pltpu.async_copy(rows_v.at[k], out.at[b0 + v], o_sem.at[k])

    def wait_out(k):
        pltpu.make_async_copy(rows_v.at[k], out.at[b0], o_sem.at[k]).wait()

    for v in range(NBUF):
        k = v % NBUF
        compute_idx(v)
        issue_gather(v, k)
        numeric(v, k)
        if v >= LAG:
            p = v - LAG
            kp = p % NBUF
            wait_gather(kp)
            issue_out(p, kp)

    def round_body(r, carry):
        vbase = r * NBUF
        for k in range(NBUF):
            v = vbase + k
            wait_out(k)
            compute_idx(v)
            issue_gather(v, k)
            numeric(v, k)
            p = v - LAG
            kp = (k + NBUF - LAG) % NBUF
            wait_gather(kp)
            issue_out(p, kp)
        return carry

    lax.fori_loop(1, BPW // NBUF, round_body, 0)

    for p in range(BPW - LAG, BPW):
        kp = p % NBUF
        wait_gather(kp)
        issue_out(p, kp)
    for k in range(NBUF):
        wait_out(k)


_mesh = plsc.VectorSubcoreMesh(core_axis_name="c", subcore_axis_name="s")

_sc_call = functools.partial(
    pl.kernel,
    out_type=jax.ShapeDtypeStruct((B, NROW, DIM), jnp.float32),
    mesh=_mesh,
    compiler_params=pltpu.CompilerParams(use_tc_tiling_on_sc=False),
    scratch_types=[
        pltpu.VMEM((BPW * NCAT + L,), jnp.int32),
        pltpu.VMEM((BPW * NPAD,), jnp.int32),
        pltpu.VMEM((NBUF, NROW, DIM), jnp.float32),
        pltpu.VMEM((BPW * NCON + L,), jnp.float32),
        pltpu.VMEM((NCON, DIM), jnp.float32),
        pltpu.VMEM((NCON, DIM), jnp.float32),
        pltpu.SemaphoreType.DMA((NBUF,)),
        pltpu.SemaphoreType.DMA((NBUF,)),
    ],
)(_body)


@jax.jit
def kernel(x_categ, x_numer, embed_table, num_weight, num_bias):
    return _sc_call(
        x_categ.astype(jnp.int32).reshape(-1),
        x_numer.astype(jnp.float32).reshape(-1),
        embed_table.astype(jnp.float32),
        num_weight.astype(jnp.float32),
        num_bias.astype(jnp.float32),
    )
